# Initial kernel scaffold; baseline (speedup 1.0000x reference)
import jax
import jax.numpy as jnp
from jax.experimental import pallas as pl


def kernel(x, embeddings):
    raise NotImplementedError("write your pallas kernel here")


# speedup vs baseline: 1.2791x; 1.2791x over previous
import functools

import jax
import jax.numpy as jnp
from jax import lax
from jax.experimental import pallas as pl
from jax.experimental.pallas import tpu as pltpu
from jax.experimental.pallas import tpu_sc as plsc

_N = 32768
_D = 256
_K = 8192
_BETA = 0.25

_BN = 256
_NB = _N // _BN

_NC = 2
_NS = 16
_NW = _NC * _NS
_BPW = _N // _NW
_CH = 256
_NCH = _BPW // _CH


def _dist_body(x_ref, emb_ref, idx_ref, minp_ref):
    xb = x_ref[...]
    em = emb_ref[...]
    sim = lax.dot_general(xb, em, (((1,), (0,)), ((), ())),
                          preferred_element_type=jnp.float32)
    x2 = jnp.sum(xb * xb, axis=1, keepdims=True)
    e2 = jnp.sum(em * em, axis=0, keepdims=True)
    dist = (x2 + e2) - 2.0 * sim
    m = jnp.min(dist, axis=1, keepdims=True)
    ids = lax.broadcasted_iota(jnp.int32, dist.shape, 1)
    first = jnp.min(jnp.where(dist <= m, ids, jnp.int32(_K)), axis=1)
    idx_ref[0, 0, :] = first
    minp_ref[0, 0, :] = jnp.broadcast_to(jnp.sum(m), (128,))


_dist_call = pl.pallas_call(
    _dist_body,
    grid=(_NB,),
    in_specs=[
        pl.BlockSpec((_BN, _D), lambda i: (i, 0)),
        pl.BlockSpec((_D, _K), lambda i: (0, 0)),
    ],
    out_specs=[
        pl.BlockSpec((1, 1, _BN), lambda i: (i, 0, 0)),
        pl.BlockSpec((1, 1, 128), lambda i: (i, 0, 0)),
    ],
    out_shape=[
        jax.ShapeDtypeStruct((_NB, 1, _BN), jnp.int32),
        jax.ShapeDtypeStruct((_NB, 1, 128), jnp.float32),
    ],
    compiler_params=pltpu.CompilerParams(
        dimension_semantics=("parallel",),
    ),
)


def _gather_body(table_hbm, idx_hbm, out_hbm, idx_v, rows_v, gsem):
    wid = lax.axis_index("s") * _NC + lax.axis_index("c")
    base = wid * _BPW
    pltpu.sync_copy(idx_hbm.at[pl.ds(base, _BPW)], idx_v)
    for j in range(_NCH):
        pltpu.async_copy(table_hbm.at[idx_v.at[pl.ds(j * _CH, _CH)]],
                         rows_v, gsem).wait()
        pltpu.sync_copy(rows_v, out_hbm.at[pl.ds(base + j * _CH, _CH)])


@functools.cache
def _gather_call():
    return functools.partial(
        pl.kernel,
        out_type=jax.ShapeDtypeStruct((_N, _D), jnp.float32),
        mesh=plsc.VectorSubcoreMesh(core_axis_name="c", subcore_axis_name="s"),
        scratch_types=[
            pltpu.VMEM((_BPW,), jnp.int32),
            pltpu.VMEM((_CH, _D), jnp.float32),
            pltpu.SemaphoreType.DMA,
        ],
    )(_gather_body)


def kernel(x, embeddings):
    idx3, minp = _dist_call(x, embeddings)
    idx = idx3.reshape(_N)
    emb_t = embeddings.T
    quantized = _gather_call()(emb_t, idx)
    loss = (1.0 + _BETA) * jnp.sum(minp[:, 0, 0]) / (_N * _D)
    return quantized, loss


# speedup vs baseline: 1.5854x; 1.2395x over previous
import functools

import jax
import jax.numpy as jnp
from jax import lax
from jax.experimental import pallas as pl
from jax.experimental.pallas import tpu as pltpu
from jax.experimental.pallas import tpu_sc as plsc

_N = 32768
_D = 256
_K = 8192
_BETA = 0.25

_BN = 256
_NB = _N // _BN

_NC = 2
_NS = 16
_NW = _NC * _NS
_BPW = _N // _NW
_CH = 256
_NCH = _BPW // _CH


def _e2_body(emb_ref, he2_ref):
    em = emb_ref[...]
    he2_ref[...] = 0.5 * jnp.sum(em * em, axis=0, keepdims=True)


_e2_call = pl.pallas_call(
    _e2_body,
    grid=(1,),
    in_specs=[pl.BlockSpec((_D, _K), lambda i: (0, 0))],
    out_specs=pl.BlockSpec((1, _K), lambda i: (0, 0)),
    out_shape=jax.ShapeDtypeStruct((1, _K), jnp.float32),
)


def _dist_body(x_ref, emb_ref, he2_ref, iota_ref, idx_ref, minp_ref):
    xb = x_ref[...]
    em = emb_ref[...]
    sim = lax.dot_general(xb, em, (((1,), (0,)), ((), ())),
                          preferred_element_type=jnp.float32)
    hx2 = 0.5 * jnp.sum(xb * xb, axis=1, keepdims=True)
    he2 = he2_ref[...]
    dist = (hx2 + he2) - sim
    m = jnp.min(dist, axis=1, keepdims=True)
    ids = jnp.broadcast_to(iota_ref[...], dist.shape)
    first = jnp.min(jnp.where(dist <= m, ids, jnp.float32(_K)), axis=1)
    idx_ref[0, 0, :] = first.astype(jnp.int32)
    minp_ref[0, 0, :] = jnp.broadcast_to(2.0 * jnp.sum(m), (128,))


_dist_call = pl.pallas_call(
    _dist_body,
    grid=(_NB,),
    in_specs=[
        pl.BlockSpec((_BN, _D), lambda i: (i, 0)),
        pl.BlockSpec((_D, _K), lambda i: (0, 0)),
        pl.BlockSpec((1, _K), lambda i: (0, 0)),
        pl.BlockSpec((1, _K), lambda i: (0, 0)),
    ],
    out_specs=[
        pl.BlockSpec((1, 1, _BN), lambda i: (i, 0, 0)),
        pl.BlockSpec((1, 1, 128), lambda i: (i, 0, 0)),
    ],
    out_shape=[
        jax.ShapeDtypeStruct((_NB, 1, _BN), jnp.int32),
        jax.ShapeDtypeStruct((_NB, 1, 128), jnp.float32),
    ],
    compiler_params=pltpu.CompilerParams(
        dimension_semantics=("parallel",),
    ),
)


def _gather_body(table_hbm, idx_hbm, out_hbm, idx_v, rows_v, gsem):
    wid = lax.axis_index("s") * _NC + lax.axis_index("c")
    base = wid * _BPW
    pltpu.sync_copy(idx_hbm.at[pl.ds(base, _BPW)], idx_v)
    for j in range(_NCH):
        pltpu.async_copy(table_hbm.at[idx_v.at[pl.ds(j * _CH, _CH)]],
                         rows_v, gsem).wait()
        pltpu.sync_copy(rows_v, out_hbm.at[pl.ds(base + j * _CH, _CH)])


@functools.cache
def _gather_call():
    return functools.partial(
        pl.kernel,
        out_type=jax.ShapeDtypeStruct((_N, _D), jnp.float32),
        mesh=plsc.VectorSubcoreMesh(core_axis_name="c", subcore_axis_name="s"),
        scratch_types=[
            pltpu.VMEM((_BPW,), jnp.int32),
            pltpu.VMEM((_CH, _D), jnp.float32),
            pltpu.SemaphoreType.DMA,
        ],
    )(_gather_body)


def kernel(x, embeddings):
    he2 = _e2_call(embeddings)
    iota = lax.broadcasted_iota(jnp.float32, (1, _K), 1)
    idx3, minp = _dist_call(x, embeddings, he2, iota)
    idx = idx3.reshape(_N)
    emb_t = embeddings.T
    quantized = _gather_call()(emb_t, idx)
    loss = (1.0 + _BETA) * jnp.sum(minp[:, 0, 0]) / (_N * _D)
    return quantized, loss
